# Initial kernel scaffold; baseline (speedup 1.0000x reference)
import jax
import jax.numpy as jnp
from jax.experimental import pallas as pl


def kernel(x, edge_index, W1, b1, W2, b2):
    raise NotImplementedError("write your pallas kernel here")


# speedup vs baseline: 17.9575x; 17.9575x over previous
import functools

import jax
import jax.numpy as jnp
from jax import lax
from jax.experimental import pallas as pl
from jax.experimental.pallas import tpu as pltpu
from jax.experimental.pallas import tpu_sc as plsc

N = 10000
NPAD = 10240
E = 320000
D_IN = 128
D_HID = 256
D_OUT = 2

NC = 2
NS = 16
NW = NC * NS
CHUNK = 128
NCH = E // CHUNK
CH_FULL = NCH // NW
CH_REM = NCH - CH_FULL * NW
ROWS_T = NPAD // NS

_mesh = plsc.VectorSubcoreMesh(core_axis_name="c", subcore_axis_name="s")


@functools.partial(
    pl.kernel,
    out_type=jax.ShapeDtypeStruct((NC, NPAD), jnp.float32),
    mesh=_mesh,
    scratch_types=[
        pltpu.VMEM((ROWS_T,), jnp.float32),
        pltpu.VMEM((CHUNK,), jnp.float32),
        pltpu.VMEM((CHUNK,), jnp.int32),
        pltpu.VMEM_SHARED((NPAD,), jnp.float32),
    ],
)
def _sc_degree(dst_e, deg_out, z_v, ones_v, dst_v, acc):
    c = lax.axis_index("c")
    s = lax.axis_index("s")
    wid = s * NC + c

    for k in range(ROWS_T // 16):
        z_v[pl.ds(16 * k, 16)] = jnp.zeros((16,), jnp.float32)
    for k in range(CHUNK // 16):
        ones_v[pl.ds(16 * k, 16)] = jnp.ones((16,), jnp.float32)
    pltpu.sync_copy(z_v, acc.at[pl.ds(s * ROWS_T, ROWS_T)])
    plsc.subcore_barrier()

    def step(cid):
        base = cid * CHUNK
        pltpu.sync_copy(dst_e.at[pl.ds(base, CHUNK)], dst_v)
        pltpu.sync_copy(ones_v, acc.at[dst_v], add=True)

    def body(g, carry):
        step(g * NW + wid)
        return carry

    lax.fori_loop(0, CH_FULL, body, 0)

    @pl.when(wid < CH_REM)
    def _():
        step(CH_FULL * NW + wid)

    plsc.subcore_barrier()
    pltpu.sync_copy(acc.at[pl.ds(s * ROWS_T, ROWS_T)],
                    deg_out.at[c, pl.ds(s * ROWS_T, ROWS_T)])


def _make_sc_agg(feat):
    @functools.partial(
        pl.kernel,
        out_type=jax.ShapeDtypeStruct((NC, NPAD, feat), jnp.float32),
        mesh=_mesh,
        scratch_types=[
            pltpu.VMEM((CHUNK,), jnp.int32),
            pltpu.VMEM((CHUNK,), jnp.int32),
            pltpu.VMEM((CHUNK, feat), jnp.float32),
            pltpu.VMEM_SHARED((NPAD, feat), jnp.float32),
        ],
    )
    def _agg(rows_hbm, src_e, dst_e, agg_out, src_v, dst_v, rows_v, acc):
        c = lax.axis_index("c")
        s = lax.axis_index("s")
        wid = s * NC + c

        pltpu.sync_copy(rows_hbm.at[pl.ds(s * ROWS_T, ROWS_T)],
                        acc.at[pl.ds(s * ROWS_T, ROWS_T)])
        plsc.subcore_barrier()

        def step(cid):
            base = cid * CHUNK
            pltpu.sync_copy(src_e.at[pl.ds(base, CHUNK)], src_v)
            pltpu.sync_copy(dst_e.at[pl.ds(base, CHUNK)], dst_v)
            pltpu.sync_copy(rows_hbm.at[src_v], rows_v)
            pltpu.sync_copy(rows_v, acc.at[dst_v], add=True)

        def body(g, carry):
            step(g * NW + wid)
            return carry

        lax.fori_loop(0, CH_FULL, body, 0)

        @pl.when(wid < CH_REM)
        def _():
            step(CH_FULL * NW + wid)

        plsc.subcore_barrier()
        pltpu.sync_copy(acc.at[pl.ds(s * ROWS_T, ROWS_T)],
                        agg_out.at[c, pl.ds(s * ROWS_T, ROWS_T)])

    return _agg


_sc_agg1 = _make_sc_agg(D_IN)


@functools.partial(
    pl.kernel,
    out_type=(jax.ShapeDtypeStruct((NC, NPAD), jnp.float32),
              jax.ShapeDtypeStruct((NC, NPAD), jnp.float32)),
    mesh=_mesh,
    scratch_types=[
        pltpu.VMEM((CHUNK,), jnp.int32),
        pltpu.VMEM((CHUNK,), jnp.int32),
        pltpu.VMEM((CHUNK,), jnp.float32),
        pltpu.VMEM((CHUNK,), jnp.float32),
        pltpu.VMEM_SHARED((NPAD,), jnp.float32),
        pltpu.VMEM_SHARED((NPAD,), jnp.float32),
    ],
)
def _sc_agg2(za, zb, src_e, dst_e, outa, outb, src_v, dst_v, ea_v, eb_v,
             acca, accb):
    c = lax.axis_index("c")
    s = lax.axis_index("s")
    wid = s * NC + c

    pltpu.sync_copy(za.at[pl.ds(s * ROWS_T, ROWS_T)],
                    acca.at[pl.ds(s * ROWS_T, ROWS_T)])
    pltpu.sync_copy(zb.at[pl.ds(s * ROWS_T, ROWS_T)],
                    accb.at[pl.ds(s * ROWS_T, ROWS_T)])
    plsc.subcore_barrier()

    def step(cid):
        base = cid * CHUNK
        pltpu.sync_copy(src_e.at[pl.ds(base, CHUNK)], src_v)
        pltpu.sync_copy(dst_e.at[pl.ds(base, CHUNK)], dst_v)
        pltpu.sync_copy(za.at[src_v], ea_v)
        pltpu.sync_copy(zb.at[src_v], eb_v)
        pltpu.sync_copy(ea_v, acca.at[dst_v], add=True)
        pltpu.sync_copy(eb_v, accb.at[dst_v], add=True)

    def body(g, carry):
        step(g * NW + wid)
        return carry

    lax.fori_loop(0, CH_FULL, body, 0)

    @pl.when(wid < CH_REM)
    def _():
        step(CH_FULL * NW + wid)

    plsc.subcore_barrier()
    pltpu.sync_copy(acca.at[pl.ds(s * ROWS_T, ROWS_T)],
                    outa.at[c, pl.ds(s * ROWS_T, ROWS_T)])
    pltpu.sync_copy(accb.at[pl.ds(s * ROWS_T, ROWS_T)],
                    outb.at[c, pl.ds(s * ROWS_T, ROWS_T)])


def _tc_scale_body(dega_ref, degb_ref, x_ref, xs_ref, dis_ref):
    deg = dega_ref[...] + degb_ref[...] + 1.0
    dis = lax.rsqrt(deg)
    dis_ref[...] = dis
    xs_ref[...] = x_ref[...] * dis


def _tc_scale(dega, degb, x):
    return pl.pallas_call(
        _tc_scale_body,
        out_shape=(
            jax.ShapeDtypeStruct((NPAD, D_IN), jnp.float32),
            jax.ShapeDtypeStruct((NPAD, 1), jnp.float32),
        ),
    )(dega, degb, x)


_RB = 2048


def _tc_mlp_body(acc_ref, xs_ref, dis_ref, w1_ref, b1_ref, w2_ref, z2s_ref):
    dis = dis_ref[...]
    a = (acc_ref[0] + acc_ref[1] - xs_ref[...]) * dis
    h = jnp.maximum(
        jnp.dot(a, w1_ref[...], preferred_element_type=jnp.float32)
        + b1_ref[...], 0.0)
    z2 = jnp.dot(h, w2_ref[...], preferred_element_type=jnp.float32)
    z2s_ref[...] = z2 * dis


def _tc_mlp(acc, xs, dis, W1, b1, W2):
    grid = (NPAD // _RB,)
    return pl.pallas_call(
        _tc_mlp_body,
        grid=grid,
        in_specs=[
            pl.BlockSpec((NC, _RB, D_IN), lambda i: (0, i, 0)),
            pl.BlockSpec((_RB, D_IN), lambda i: (i, 0)),
            pl.BlockSpec((_RB, 1), lambda i: (i, 0)),
            pl.BlockSpec((D_IN, D_HID), lambda i: (0, 0)),
            pl.BlockSpec((1, D_HID), lambda i: (0, 0)),
            pl.BlockSpec((D_HID, D_OUT), lambda i: (0, 0)),
        ],
        out_specs=pl.BlockSpec((_RB, D_OUT), lambda i: (i, 0)),
        out_shape=jax.ShapeDtypeStruct((NPAD, D_OUT), jnp.float32),
    )(acc, xs, dis, W1, b1, W2)


def _tc_final_body(a0_ref, a1_ref, b0_ref, b1c_ref, za_ref, zb_ref,
                   dis_ref, b2_ref, oa_ref, ob_ref):
    dis = dis_ref[...]
    oa_ref[...] = ((a0_ref[...] + a1_ref[...] - za_ref[...]) * dis
                   + b2_ref[0:1, 0:1])
    ob_ref[...] = ((b0_ref[...] + b1c_ref[...] - zb_ref[...]) * dis
                   + b2_ref[0:1, 1:2])


def _tc_final(acc2a, acc2b, za, zb, dis, b2):
    return pl.pallas_call(
        _tc_final_body,
        out_shape=(jax.ShapeDtypeStruct((N, 1), jnp.float32),
                   jax.ShapeDtypeStruct((N, 1), jnp.float32)),
    )(acc2a[0, :N, None], acc2a[1, :N, None],
      acc2b[0, :N, None], acc2b[1, :N, None],
      za[:N, None], zb[:N, None], dis[:N], b2.reshape(1, D_OUT))


def kernel(x, edge_index, W1, b1, W2, b2):
    ei = edge_index.astype(jnp.int32)
    src_e = ei[0]
    dst_e = ei[1]
    xp = jnp.pad(x, ((0, NPAD - N), (0, 0)))
    deg2 = _sc_degree(dst_e)
    xs, dis = _tc_scale(deg2[0, :, None], deg2[1, :, None], xp)
    acc = _sc_agg1(xs, src_e, dst_e)
    z2s = _tc_mlp(acc, xs, dis, W1, b1.reshape(1, D_HID), W2)
    acc2a, acc2b = _sc_agg2(z2s[:, 0], z2s[:, 1], src_e, dst_e)
    oa, ob = _tc_final(acc2a, acc2b, z2s[:, 0], z2s[:, 1], dis, b2)
    return jnp.concatenate([oa, ob], axis=1)


# speedup vs baseline: 31.9497x; 1.7792x over previous
import functools

import jax
import jax.numpy as jnp
from jax import lax
from jax.experimental import pallas as pl
from jax.experimental.pallas import tpu as pltpu
from jax.experimental.pallas import tpu_sc as plsc

N = 10000
NPAD = 10240
E = 320000
D_IN = 128
D_HID = 256
D_OUT = 2

NC = 2
NS = 16
NW = NC * NS
CHUNK = 128
NCH = E // CHUNK
CH_FULL = NCH // NW
CH_REM = NCH - CH_FULL * NW
ROWS_T = NPAD // NS

_mesh = plsc.VectorSubcoreMesh(core_axis_name="c", subcore_axis_name="s")


@functools.partial(
    pl.kernel,
    out_type=jax.ShapeDtypeStruct((NC, NPAD), jnp.float32),
    mesh=_mesh,
    scratch_types=[
        pltpu.VMEM((ROWS_T,), jnp.float32),
        pltpu.VMEM((CHUNK,), jnp.float32),
        pltpu.VMEM((2, CHUNK), jnp.int32),
        pltpu.VMEM_SHARED((NPAD,), jnp.float32),
        pltpu.SemaphoreType.DMA((2,)),
        pltpu.SemaphoreType.DMA((2,)),
    ],
)
def _sc_degree(dst_e, deg_out, z_v, ones_v, dst_v, acc, sem_i, sem_s):
    c = lax.axis_index("c")
    s = lax.axis_index("s")
    wid = s * NC + c

    for k in range(ROWS_T // 16):
        z_v[pl.ds(16 * k, 16)] = jnp.zeros((16,), jnp.float32)
    for k in range(CHUNK // 16):
        ones_v[pl.ds(16 * k, 16)] = jnp.ones((16,), jnp.float32)
    pltpu.sync_copy(z_v, acc.at[pl.ds(s * ROWS_T, ROWS_T)])
    plsc.subcore_barrier()

    def idx_desc(g, b):
        base = (g * NW + wid) * CHUNK
        return pltpu.make_async_copy(
            dst_e.at[pl.ds(base, CHUNK)], dst_v.at[b], sem_i.at[b])

    def scat_desc(b):
        return pltpu.make_async_copy(ones_v, acc.at[dst_v.at[b]], sem_s.at[b])

    idx_desc(0, 0).start()

    @pl.loop(0, CH_FULL, step=2)
    def _pipe(g):
        for b in range(2):
            gg = g + b

            @pl.when(gg + 1 < CH_FULL)
            def _():
                idx_desc(gg + 1, 1 - b).start()

            idx_desc(gg, b).wait()

            @pl.when(gg >= 2)
            def _():
                scat_desc(b).wait()

            pltpu.async_copy(ones_v, acc.at[dst_v.at[b]], sem_s.at[b],
                             add=True)

    scat_desc(0).wait()
    scat_desc(1).wait()

    @pl.when(wid < CH_REM)
    def _rem():
        base = (CH_FULL * NW + wid) * CHUNK
        pltpu.sync_copy(dst_e.at[pl.ds(base, CHUNK)], dst_v.at[0])
        pltpu.sync_copy(ones_v, acc.at[dst_v.at[0]], add=True)

    plsc.subcore_barrier()
    pltpu.sync_copy(acc.at[pl.ds(s * ROWS_T, ROWS_T)],
                    deg_out.at[c, pl.ds(s * ROWS_T, ROWS_T)])


@functools.partial(
    pl.kernel,
    out_type=jax.ShapeDtypeStruct((NC, NPAD, D_IN), jnp.float32),
    mesh=_mesh,
    scratch_types=[
        pltpu.VMEM((2, CHUNK), jnp.int32),
        pltpu.VMEM((2, CHUNK), jnp.int32),
        pltpu.VMEM((2, CHUNK, D_IN), jnp.float32),
        pltpu.VMEM_SHARED((NPAD, D_IN), jnp.float32),
        pltpu.SemaphoreType.DMA((2,)),
        pltpu.SemaphoreType.DMA((2,)),
        pltpu.SemaphoreType.DMA((2,)),
    ],
)
def _sc_agg1(rows_hbm, src_e, dst_e, agg_out, src_v, dst_v, rows_v, acc,
             sem_i, sem_g, sem_s):
    c = lax.axis_index("c")
    s = lax.axis_index("s")
    wid = s * NC + c

    pltpu.sync_copy(rows_hbm.at[pl.ds(s * ROWS_T, ROWS_T)],
                    acc.at[pl.ds(s * ROWS_T, ROWS_T)])
    plsc.subcore_barrier()

    def idx_descs(g, b):
        base = (g * NW + wid) * CHUNK
        return (pltpu.make_async_copy(src_e.at[pl.ds(base, CHUNK)],
                                      src_v.at[b], sem_i.at[b]),
                pltpu.make_async_copy(dst_e.at[pl.ds(base, CHUNK)],
                                      dst_v.at[b], sem_i.at[b]))

    def gather_desc(b):
        return pltpu.make_async_copy(rows_hbm.at[src_v.at[b]], rows_v.at[b],
                                     sem_g.at[b])

    def scat_desc(b):
        return pltpu.make_async_copy(rows_v.at[b], acc.at[dst_v.at[b]],
                                     sem_s.at[b])

    for d in idx_descs(0, 0):
        d.start()

    @pl.loop(0, CH_FULL, step=2)
    def _pipe(g):
        for b in range(2):
            gg = g + b

            @pl.when(gg + 1 < CH_FULL)
            def _():
                for d in idx_descs(gg + 1, 1 - b):
                    d.start()

            for d in idx_descs(gg, b):
                d.wait()

            @pl.when(gg >= 2)
            def _():
                scat_desc(b).wait()

            gather_desc(b).start()
            gather_desc(b).wait()
            pltpu.async_copy(rows_v.at[b], acc.at[dst_v.at[b]], sem_s.at[b],
                             add=True)

    scat_desc(0).wait()
    scat_desc(1).wait()

    @pl.when(wid < CH_REM)
    def _rem():
        base = (CH_FULL * NW + wid) * CHUNK
        pltpu.sync_copy(src_e.at[pl.ds(base, CHUNK)], src_v.at[0])
        pltpu.sync_copy(dst_e.at[pl.ds(base, CHUNK)], dst_v.at[0])
        pltpu.sync_copy(rows_hbm.at[src_v.at[0]], rows_v.at[0])
        pltpu.sync_copy(rows_v.at[0], acc.at[dst_v.at[0]], add=True)

    plsc.subcore_barrier()
    pltpu.sync_copy(acc.at[pl.ds(s * ROWS_T, ROWS_T)],
                    agg_out.at[c, pl.ds(s * ROWS_T, ROWS_T)])


@functools.partial(
    pl.kernel,
    out_type=(jax.ShapeDtypeStruct((NC, NPAD), jnp.float32),
              jax.ShapeDtypeStruct((NC, NPAD), jnp.float32)),
    mesh=_mesh,
    scratch_types=[
        pltpu.VMEM((2, CHUNK), jnp.int32),
        pltpu.VMEM((2, CHUNK), jnp.int32),
        pltpu.VMEM((2, CHUNK), jnp.float32),
        pltpu.VMEM((2, CHUNK), jnp.float32),
        pltpu.VMEM_SHARED((NPAD,), jnp.float32),
        pltpu.VMEM_SHARED((NPAD,), jnp.float32),
        pltpu.SemaphoreType.DMA((2,)),
        pltpu.SemaphoreType.DMA((2,)),
        pltpu.SemaphoreType.DMA((2,)),
    ],
)
def _sc_agg2(za, zb, src_e, dst_e, outa, outb, src_v, dst_v, ea_v, eb_v,
             acca, accb, sem_i, sem_g, sem_s):
    c = lax.axis_index("c")
    s = lax.axis_index("s")
    wid = s * NC + c

    pltpu.sync_copy(za.at[pl.ds(s * ROWS_T, ROWS_T)],
                    acca.at[pl.ds(s * ROWS_T, ROWS_T)])
    pltpu.sync_copy(zb.at[pl.ds(s * ROWS_T, ROWS_T)],
                    accb.at[pl.ds(s * ROWS_T, ROWS_T)])
    plsc.subcore_barrier()

    def idx_descs(g, b):
        base = (g * NW + wid) * CHUNK
        return (pltpu.make_async_copy(src_e.at[pl.ds(base, CHUNK)],
                                      src_v.at[b], sem_i.at[b]),
                pltpu.make_async_copy(dst_e.at[pl.ds(base, CHUNK)],
                                      dst_v.at[b], sem_i.at[b]))

    def gather_descs(b):
        return (pltpu.make_async_copy(za.at[src_v.at[b]], ea_v.at[b],
                                      sem_g.at[b]),
                pltpu.make_async_copy(zb.at[src_v.at[b]], eb_v.at[b],
                                      sem_g.at[b]))

    def scat_descs(b):
        return (pltpu.make_async_copy(ea_v.at[b], acca.at[dst_v.at[b]],
                                      sem_s.at[b]),
                pltpu.make_async_copy(eb_v.at[b], accb.at[dst_v.at[b]],
                                      sem_s.at[b]))

    for d in idx_descs(0, 0):
        d.start()

    @pl.loop(0, CH_FULL, step=2)
    def _pipe(g):
        for b in range(2):
            gg = g + b

            @pl.when(gg + 1 < CH_FULL)
            def _():
                for d in idx_descs(gg + 1, 1 - b):
                    d.start()

            for d in idx_descs(gg, b):
                d.wait()

            @pl.when(gg >= 2)
            def _():
                for d in scat_descs(b):
                    d.wait()

            for d in gather_descs(b):
                d.start()
            for d in gather_descs(b):
                d.wait()
            pltpu.async_copy(ea_v.at[b], acca.at[dst_v.at[b]], sem_s.at[b],
                             add=True)
            pltpu.async_copy(eb_v.at[b], accb.at[dst_v.at[b]], sem_s.at[b],
                             add=True)

    for b in range(2):
        for d in scat_descs(b):
            d.wait()

    @pl.when(wid < CH_REM)
    def _rem():
        base = (CH_FULL * NW + wid) * CHUNK
        pltpu.sync_copy(src_e.at[pl.ds(base, CHUNK)], src_v.at[0])
        pltpu.sync_copy(dst_e.at[pl.ds(base, CHUNK)], dst_v.at[0])
        pltpu.sync_copy(za.at[src_v.at[0]], ea_v.at[0])
        pltpu.sync_copy(zb.at[src_v.at[0]], eb_v.at[0])
        pltpu.sync_copy(ea_v.at[0], acca.at[dst_v.at[0]], add=True)
        pltpu.sync_copy(eb_v.at[0], accb.at[dst_v.at[0]], add=True)

    plsc.subcore_barrier()
    pltpu.sync_copy(acca.at[pl.ds(s * ROWS_T, ROWS_T)],
                    outa.at[c, pl.ds(s * ROWS_T, ROWS_T)])
    pltpu.sync_copy(accb.at[pl.ds(s * ROWS_T, ROWS_T)],
                    outb.at[c, pl.ds(s * ROWS_T, ROWS_T)])


def _tc_scale_body(dega_ref, degb_ref, x_ref, xs_ref, dis_ref):
    deg = dega_ref[...] + degb_ref[...] + 1.0
    dis = lax.rsqrt(deg)
    dis_ref[...] = dis
    xs_ref[...] = x_ref[...] * dis


def _tc_scale(dega, degb, x):
    return pl.pallas_call(
        _tc_scale_body,
        out_shape=(
            jax.ShapeDtypeStruct((NPAD, D_IN), jnp.float32),
            jax.ShapeDtypeStruct((NPAD, 1), jnp.float32),
        ),
    )(dega, degb, x)


_RB = 2048


def _tc_mlp_body(acc_ref, xs_ref, dis_ref, w1_ref, b1_ref, w2_ref, z2s_ref):
    dis = dis_ref[...]
    a = (acc_ref[0] + acc_ref[1] - xs_ref[...]) * dis
    h = jnp.maximum(
        jnp.dot(a, w1_ref[...], preferred_element_type=jnp.float32)
        + b1_ref[...], 0.0)
    z2 = jnp.dot(h, w2_ref[...], preferred_element_type=jnp.float32)
    z2s_ref[...] = z2 * dis


def _tc_mlp(acc, xs, dis, W1, b1, W2):
    grid = (NPAD // _RB,)
    return pl.pallas_call(
        _tc_mlp_body,
        grid=grid,
        in_specs=[
            pl.BlockSpec((NC, _RB, D_IN), lambda i: (0, i, 0)),
            pl.BlockSpec((_RB, D_IN), lambda i: (i, 0)),
            pl.BlockSpec((_RB, 1), lambda i: (i, 0)),
            pl.BlockSpec((D_IN, D_HID), lambda i: (0, 0)),
            pl.BlockSpec((1, D_HID), lambda i: (0, 0)),
            pl.BlockSpec((D_HID, D_OUT), lambda i: (0, 0)),
        ],
        out_specs=pl.BlockSpec((_RB, D_OUT), lambda i: (i, 0)),
        out_shape=jax.ShapeDtypeStruct((NPAD, D_OUT), jnp.float32),
    )(acc, xs, dis, W1, b1, W2)


def _tc_final_body(a0_ref, a1_ref, b0_ref, b1c_ref, za_ref, zb_ref,
                   dis_ref, b2_ref, oa_ref, ob_ref):
    dis = dis_ref[...]
    oa_ref[...] = ((a0_ref[...] + a1_ref[...] - za_ref[...]) * dis
                   + b2_ref[0:1, 0:1])
    ob_ref[...] = ((b0_ref[...] + b1c_ref[...] - zb_ref[...]) * dis
                   + b2_ref[0:1, 1:2])


def _tc_final(acc2a, acc2b, za, zb, dis, b2):
    return pl.pallas_call(
        _tc_final_body,
        out_shape=(jax.ShapeDtypeStruct((N, 1), jnp.float32),
                   jax.ShapeDtypeStruct((N, 1), jnp.float32)),
    )(acc2a[0, :N, None], acc2a[1, :N, None],
      acc2b[0, :N, None], acc2b[1, :N, None],
      za[:N, None], zb[:N, None], dis[:N], b2.reshape(1, D_OUT))


def kernel(x, edge_index, W1, b1, W2, b2):
    ei = edge_index.astype(jnp.int32)
    src_e = ei[0]
    dst_e = ei[1]
    xp = jnp.pad(x, ((0, NPAD - N), (0, 0)))
    deg2 = _sc_degree(dst_e)
    xs, dis = _tc_scale(deg2[0, :, None], deg2[1, :, None], xp)
    acc = _sc_agg1(xs, src_e, dst_e)
    z2s = _tc_mlp(acc, xs, dis, W1, b1.reshape(1, D_HID), W2)
    acc2a, acc2b = _sc_agg2(z2s[:, 0], z2s[:, 1], src_e, dst_e)
    oa, ob = _tc_final(acc2a, acc2b, z2s[:, 0], z2s[:, 1], dis, b2)
    return jnp.concatenate([oa, ob], axis=1)


# speedup vs baseline: 35.2844x; 1.1044x over previous
import functools

import jax
import jax.numpy as jnp
from jax import lax
from jax.experimental import pallas as pl
from jax.experimental.pallas import tpu as pltpu
from jax.experimental.pallas import tpu_sc as plsc

N = 10000
NPAD = 10240
E = 320000
D_IN = 128
D_HID = 256
D_OUT = 2

NC = 2
NS = 16
NW = NC * NS
CHUNK = 128
NCH = E // CHUNK
CH_FULL = NCH // NW
CH_REM = NCH - CH_FULL * NW
ROWS_T = NPAD // NS
NBUF = 4
CH_MAIN = (CH_FULL - 2) // NBUF * NBUF
assert CH_FULL - CH_MAIN == 2

_mesh = plsc.VectorSubcoreMesh(core_axis_name="c", subcore_axis_name="s")


@functools.partial(
    pl.kernel,
    out_type=jax.ShapeDtypeStruct((NC, NPAD), jnp.float32),
    mesh=_mesh,
    scratch_types=[
        pltpu.VMEM((ROWS_T,), jnp.float32),
        pltpu.VMEM((CHUNK,), jnp.float32),
        pltpu.VMEM((NBUF, CHUNK), jnp.int32),
        pltpu.VMEM_SHARED((NPAD,), jnp.float32),
        pltpu.SemaphoreType.DMA((NBUF,)),
        pltpu.SemaphoreType.DMA((NBUF,)),
    ],
)
def _sc_degree(dst_e, deg_out, z_v, ones_v, dst_v, acc, sem_i, sem_s):
    c = lax.axis_index("c")
    s = lax.axis_index("s")
    wid = s * NC + c

    for k in range(ROWS_T // 16):
        z_v[pl.ds(16 * k, 16)] = jnp.zeros((16,), jnp.float32)
    for k in range(CHUNK // 16):
        ones_v[pl.ds(16 * k, 16)] = jnp.ones((16,), jnp.float32)
    pltpu.sync_copy(z_v, acc.at[pl.ds(s * ROWS_T, ROWS_T)])
    plsc.subcore_barrier()

    def idx_desc(g, b):
        base = (g * NW + wid) * CHUNK
        return pltpu.make_async_copy(
            dst_e.at[pl.ds(base, CHUNK)], dst_v.at[b], sem_i.at[b])

    def scat_desc(b):
        return pltpu.make_async_copy(ones_v, acc.at[dst_v.at[b]], sem_s.at[b])

    idx_desc(0, 0).start()
    idx_desc(1, 1).start()

    @pl.loop(0, CH_MAIN, step=NBUF)
    def _pipe(g):
        for u in range(NBUF):
            gg = g + u
            bp = (u + 2) % NBUF

            @pl.when(gg >= 2)
            def _():
                scat_desc(bp).wait()

            idx_desc(gg + 2, bp).start()
            idx_desc(gg, u).wait()
            pltpu.async_copy(ones_v, acc.at[dst_v.at[u]], sem_s.at[u],
                             add=True)

    for gg in (CH_MAIN, CH_MAIN + 1):
        b = gg % NBUF
        scat_desc((gg + 2) % NBUF).wait()
        idx_desc(gg, b).wait()
        pltpu.async_copy(ones_v, acc.at[dst_v.at[b]], sem_s.at[b], add=True)
    scat_desc(CH_MAIN % NBUF).wait()
    scat_desc((CH_MAIN + 1) % NBUF).wait()

    @pl.when(wid < CH_REM)
    def _rem():
        base = (CH_FULL * NW + wid) * CHUNK
        pltpu.sync_copy(dst_e.at[pl.ds(base, CHUNK)], dst_v.at[0])
        pltpu.sync_copy(ones_v, acc.at[dst_v.at[0]], add=True)

    plsc.subcore_barrier()
    pltpu.sync_copy(acc.at[pl.ds(s * ROWS_T, ROWS_T)],
                    deg_out.at[c, pl.ds(s * ROWS_T, ROWS_T)])


NSPL = 4
SUB = CHUNK // NSPL


@functools.partial(
    pl.kernel,
    out_type=jax.ShapeDtypeStruct((NC, NPAD, D_IN), jnp.float32),
    mesh=_mesh,
    scratch_types=[
        pltpu.VMEM((NBUF, CHUNK), jnp.int32),
        pltpu.VMEM((NBUF, CHUNK), jnp.int32),
        pltpu.VMEM((2, CHUNK, D_IN), jnp.float32),
        pltpu.VMEM_SHARED((NPAD, D_IN), jnp.float32),
        pltpu.SemaphoreType.DMA((NBUF,)),
        pltpu.SemaphoreType.DMA((2,)),
        pltpu.SemaphoreType.DMA((2,)),
    ],
)
def _sc_agg1(rows_hbm, src_e, dst_e, agg_out, src_v, dst_v, rows_v, acc,
             sem_i, sem_g, sem_s):
    c = lax.axis_index("c")
    s = lax.axis_index("s")
    wid = s * NC + c

    pltpu.sync_copy(rows_hbm.at[pl.ds(s * ROWS_T, ROWS_T)],
                    acc.at[pl.ds(s * ROWS_T, ROWS_T)])
    plsc.subcore_barrier()

    def idx_descs(g, b):
        base = (g * NW + wid) * CHUNK
        return (pltpu.make_async_copy(src_e.at[pl.ds(base, CHUNK)],
                                      src_v.at[b], sem_i.at[b]),
                pltpu.make_async_copy(dst_e.at[pl.ds(base, CHUNK)],
                                      dst_v.at[b], sem_i.at[b]))

    def sub_gather_descs(ib, rb):
        return tuple(
            pltpu.make_async_copy(
                rows_hbm.at[src_v.at[ib, pl.ds(h * SUB, SUB)]],
                rows_v.at[rb, pl.ds(h * SUB, SUB)],
                sem_g.at[rb])
            for h in range(NSPL))

    def scat_desc(ib, rb):
        return pltpu.make_async_copy(rows_v.at[rb], acc.at[dst_v.at[ib]],
                                     sem_s.at[rb])

    for d in idx_descs(0, 0):
        d.start()
    for d in idx_descs(1, 1):
        d.start()

    @pl.loop(0, CH_MAIN, step=NBUF)
    def _pipe(g):
        for u in range(NBUF):
            gg = g + u
            rb = u % 2
            ip = (u + 2) % NBUF

            @pl.when(gg >= 2)
            def _():
                scat_desc(ip, rb).wait()

            for d in idx_descs(gg + 2, ip):
                d.start()
            for d in idx_descs(gg, u):
                d.wait()
            for d in sub_gather_descs(u, rb):
                d.start()
            for d in sub_gather_descs(u, rb):
                d.wait()
            pltpu.async_copy(rows_v.at[rb], acc.at[dst_v.at[u]],
                             sem_s.at[rb], add=True)

    for gg in (CH_MAIN, CH_MAIN + 1):
        ib = gg % NBUF
        rb = gg % 2
        scat_desc((gg + 2) % NBUF, rb).wait()
        for d in idx_descs(gg, ib):
            d.wait()
        for d in sub_gather_descs(ib, rb):
            d.start()
        for d in sub_gather_descs(ib, rb):
            d.wait()
        pltpu.async_copy(rows_v.at[rb], acc.at[dst_v.at[ib]], sem_s.at[rb],
                         add=True)
    scat_desc(CH_MAIN % NBUF, CH_MAIN % 2).wait()
    scat_desc((CH_MAIN + 1) % NBUF, (CH_MAIN + 1) % 2).wait()

    @pl.when(wid < CH_REM)
    def _rem():
        base = (CH_FULL * NW + wid) * CHUNK
        pltpu.sync_copy(src_e.at[pl.ds(base, CHUNK)], src_v.at[0])
        pltpu.sync_copy(dst_e.at[pl.ds(base, CHUNK)], dst_v.at[0])
        pltpu.sync_copy(rows_hbm.at[src_v.at[0]], rows_v.at[0])
        pltpu.sync_copy(rows_v.at[0], acc.at[dst_v.at[0]], add=True)

    plsc.subcore_barrier()
    pltpu.sync_copy(acc.at[pl.ds(s * ROWS_T, ROWS_T)],
                    agg_out.at[c, pl.ds(s * ROWS_T, ROWS_T)])


@functools.partial(
    pl.kernel,
    out_type=(jax.ShapeDtypeStruct((NC, NPAD), jnp.float32),
              jax.ShapeDtypeStruct((NC, NPAD), jnp.float32)),
    mesh=_mesh,
    scratch_types=[
        pltpu.VMEM((NBUF, CHUNK), jnp.int32),
        pltpu.VMEM((NBUF, CHUNK), jnp.int32),
        pltpu.VMEM((NBUF, CHUNK), jnp.float32),
        pltpu.VMEM((NBUF, CHUNK), jnp.float32),
        pltpu.VMEM_SHARED((NPAD,), jnp.float32),
        pltpu.VMEM_SHARED((NPAD,), jnp.float32),
        pltpu.SemaphoreType.DMA((NBUF,)),
        pltpu.SemaphoreType.DMA((NBUF,)),
        pltpu.SemaphoreType.DMA((NBUF,)),
    ],
)
def _sc_agg2(za, zb, src_e, dst_e, outa, outb, src_v, dst_v, ea_v, eb_v,
             acca, accb, sem_i, sem_g, sem_s):
    c = lax.axis_index("c")
    s = lax.axis_index("s")
    wid = s * NC + c

    pltpu.sync_copy(za.at[pl.ds(s * ROWS_T, ROWS_T)],
                    acca.at[pl.ds(s * ROWS_T, ROWS_T)])
    pltpu.sync_copy(zb.at[pl.ds(s * ROWS_T, ROWS_T)],
                    accb.at[pl.ds(s * ROWS_T, ROWS_T)])
    plsc.subcore_barrier()

    def idx_descs(g, b):
        base = (g * NW + wid) * CHUNK
        return (pltpu.make_async_copy(src_e.at[pl.ds(base, CHUNK)],
                                      src_v.at[b], sem_i.at[b]),
                pltpu.make_async_copy(dst_e.at[pl.ds(base, CHUNK)],
                                      dst_v.at[b], sem_i.at[b]))

    def gather_descs(b):
        return (pltpu.make_async_copy(za.at[src_v.at[b]], ea_v.at[b],
                                      sem_g.at[b]),
                pltpu.make_async_copy(zb.at[src_v.at[b]], eb_v.at[b],
                                      sem_g.at[b]))

    def scat_descs(b):
        return (pltpu.make_async_copy(ea_v.at[b], acca.at[dst_v.at[b]],
                                      sem_s.at[b]),
                pltpu.make_async_copy(eb_v.at[b], accb.at[dst_v.at[b]],
                                      sem_s.at[b]))

    def scat_start(b):
        pltpu.async_copy(ea_v.at[b], acca.at[dst_v.at[b]], sem_s.at[b],
                         add=True)
        pltpu.async_copy(eb_v.at[b], accb.at[dst_v.at[b]], sem_s.at[b],
                         add=True)

    for d in idx_descs(0, 0):
        d.start()
    for d in idx_descs(1, 1):
        d.start()

    @pl.loop(0, CH_MAIN, step=NBUF)
    def _pipe(g):
        for u in range(NBUF):
            gg = g + u
            bp = (u + 2) % NBUF
            bm = (u - 1) % NBUF

            @pl.when(gg >= 2)
            def _():
                for d in scat_descs(bp):
                    d.wait()

            for d in idx_descs(gg + 2, bp):
                d.start()
            for d in idx_descs(gg, u):
                d.wait()
            for d in gather_descs(u):
                d.start()

            @pl.when(gg >= 1)
            def _():
                for d in gather_descs(bm):
                    d.wait()
                scat_start(bm)

    for gg in (CH_MAIN, CH_MAIN + 1):
        b = gg % NBUF
        bm = (gg - 1) % NBUF
        for d in scat_descs((gg + 2) % NBUF):
            d.wait()
        for d in idx_descs(gg, b):
            d.wait()
        for d in gather_descs(b):
            d.start()
        for d in gather_descs(bm):
            d.wait()
        scat_start(bm)
    _bl = (CH_MAIN + 1) % NBUF
    for d in gather_descs(_bl):
        d.wait()
    scat_start(_bl)
    for d in scat_descs(CH_MAIN % NBUF):
        d.wait()
    for d in scat_descs(_bl):
        d.wait()

    @pl.when(wid < CH_REM)
    def _rem():
        base = (CH_FULL * NW + wid) * CHUNK
        pltpu.sync_copy(src_e.at[pl.ds(base, CHUNK)], src_v.at[0])
        pltpu.sync_copy(dst_e.at[pl.ds(base, CHUNK)], dst_v.at[0])
        pltpu.sync_copy(za.at[src_v.at[0]], ea_v.at[0])
        pltpu.sync_copy(zb.at[src_v.at[0]], eb_v.at[0])
        pltpu.sync_copy(ea_v.at[0], acca.at[dst_v.at[0]], add=True)
        pltpu.sync_copy(eb_v.at[0], accb.at[dst_v.at[0]], add=True)

    plsc.subcore_barrier()
    pltpu.sync_copy(acca.at[pl.ds(s * ROWS_T, ROWS_T)],
                    outa.at[c, pl.ds(s * ROWS_T, ROWS_T)])
    pltpu.sync_copy(accb.at[pl.ds(s * ROWS_T, ROWS_T)],
                    outb.at[c, pl.ds(s * ROWS_T, ROWS_T)])


def _tc_scale_body(dega_ref, degb_ref, x_ref, xs_ref, dis_ref):
    deg = dega_ref[...] + degb_ref[...] + 1.0
    dis = lax.rsqrt(deg)
    dis_ref[...] = dis
    xs_ref[...] = x_ref[...] * dis


def _tc_scale(dega, degb, x):
    return pl.pallas_call(
        _tc_scale_body,
        out_shape=(
            jax.ShapeDtypeStruct((NPAD, D_IN), jnp.float32),
            jax.ShapeDtypeStruct((NPAD, 1), jnp.float32),
        ),
    )(dega, degb, x)


_RB = 2048


def _tc_mlp_body(acc_ref, xs_ref, dis_ref, w1_ref, b1_ref, w2_ref, z2s_ref):
    dis = dis_ref[...]
    a = (acc_ref[0] + acc_ref[1] - xs_ref[...]) * dis
    h = jnp.maximum(
        jnp.dot(a, w1_ref[...], preferred_element_type=jnp.float32)
        + b1_ref[...], 0.0)
    z2 = jnp.dot(h, w2_ref[...], preferred_element_type=jnp.float32)
    z2s_ref[...] = z2 * dis


def _tc_mlp(acc, xs, dis, W1, b1, W2):
    grid = (NPAD // _RB,)
    return pl.pallas_call(
        _tc_mlp_body,
        grid=grid,
        in_specs=[
            pl.BlockSpec((NC, _RB, D_IN), lambda i: (0, i, 0)),
            pl.BlockSpec((_RB, D_IN), lambda i: (i, 0)),
            pl.BlockSpec((_RB, 1), lambda i: (i, 0)),
            pl.BlockSpec((D_IN, D_HID), lambda i: (0, 0)),
            pl.BlockSpec((1, D_HID), lambda i: (0, 0)),
            pl.BlockSpec((D_HID, D_OUT), lambda i: (0, 0)),
        ],
        out_specs=pl.BlockSpec((_RB, D_OUT), lambda i: (i, 0)),
        out_shape=jax.ShapeDtypeStruct((NPAD, D_OUT), jnp.float32),
    )(acc, xs, dis, W1, b1, W2)


def _tc_final_body(a0_ref, a1_ref, b0_ref, b1c_ref, za_ref, zb_ref,
                   dis_ref, b2_ref, oa_ref, ob_ref):
    dis = dis_ref[...]
    oa_ref[...] = ((a0_ref[...] + a1_ref[...] - za_ref[...]) * dis
                   + b2_ref[0:1, 0:1])
    ob_ref[...] = ((b0_ref[...] + b1c_ref[...] - zb_ref[...]) * dis
                   + b2_ref[0:1, 1:2])


def _tc_final(acc2a, acc2b, za, zb, dis, b2):
    return pl.pallas_call(
        _tc_final_body,
        out_shape=(jax.ShapeDtypeStruct((N, 1), jnp.float32),
                   jax.ShapeDtypeStruct((N, 1), jnp.float32)),
    )(acc2a[0, :N, None], acc2a[1, :N, None],
      acc2b[0, :N, None], acc2b[1, :N, None],
      za[:N, None], zb[:N, None], dis[:N], b2.reshape(1, D_OUT))


def kernel(x, edge_index, W1, b1, W2, b2):
    ei = edge_index.astype(jnp.int32)
    src_e = ei[0]
    dst_e = ei[1]
    xp = jnp.pad(x, ((0, NPAD - N), (0, 0)))
    deg2 = _sc_degree(dst_e)
    xs, dis = _tc_scale(deg2[0, :, None], deg2[1, :, None], xp)
    acc = _sc_agg1(xs, src_e, dst_e)
    z2s = _tc_mlp(acc, xs, dis, W1, b1.reshape(1, D_HID), W2)
    acc2a, acc2b = _sc_agg2(z2s[:, 0], z2s[:, 1], src_e, dst_e)
    oa, ob = _tc_final(acc2a, acc2b, z2s[:, 0], z2s[:, 1], dis, b2)
    return jnp.concatenate([oa, ob], axis=1)


# speedup vs baseline: 35.4183x; 1.0038x over previous
import functools

import jax
import jax.numpy as jnp
from jax import lax
from jax.experimental import pallas as pl
from jax.experimental.pallas import tpu as pltpu
from jax.experimental.pallas import tpu_sc as plsc

N = 10000
NPAD = 10240
E = 320000
D_IN = 128
D_HID = 256
D_OUT = 2

NC = 2
NS = 16
NW = NC * NS
CHUNK = 128
NCH = E // CHUNK
CH_FULL = NCH // NW
CH_REM = NCH - CH_FULL * NW
ROWS_T = NPAD // NS
NBUF = 4
CH_MAIN = (CH_FULL - 2) // NBUF * NBUF
assert CH_FULL - CH_MAIN == 2

_mesh = plsc.VectorSubcoreMesh(core_axis_name="c", subcore_axis_name="s")


@functools.partial(
    pl.kernel,
    out_type=jax.ShapeDtypeStruct((NC, NPAD), jnp.float32),
    mesh=_mesh,
    scratch_types=[
        pltpu.VMEM((ROWS_T,), jnp.float32),
        pltpu.VMEM((CHUNK,), jnp.float32),
        pltpu.VMEM((NBUF, CHUNK), jnp.int32),
        pltpu.VMEM_SHARED((NPAD,), jnp.float32),
        pltpu.SemaphoreType.DMA((NBUF,)),
        pltpu.SemaphoreType.DMA((NBUF,)),
    ],
)
def _sc_degree(dst_e, deg_out, z_v, ones_v, dst_v, acc, sem_i, sem_s):
    c = lax.axis_index("c")
    s = lax.axis_index("s")
    wid = s * NC + c

    for k in range(ROWS_T // 16):
        z_v[pl.ds(16 * k, 16)] = jnp.zeros((16,), jnp.float32)
    for k in range(CHUNK // 16):
        ones_v[pl.ds(16 * k, 16)] = jnp.ones((16,), jnp.float32)
    pltpu.sync_copy(z_v, acc.at[pl.ds(s * ROWS_T, ROWS_T)])
    plsc.subcore_barrier()

    def idx_desc(g, b):
        base = (g * NW + wid) * CHUNK
        return pltpu.make_async_copy(
            dst_e.at[pl.ds(base, CHUNK)], dst_v.at[b], sem_i.at[b])

    def scat_desc(b):
        return pltpu.make_async_copy(ones_v, acc.at[dst_v.at[b]], sem_s.at[b])

    idx_desc(0, 0).start()
    idx_desc(1, 1).start()

    @pl.loop(0, CH_MAIN, step=NBUF)
    def _pipe(g):
        for u in range(NBUF):
            gg = g + u
            bp = (u + 2) % NBUF

            @pl.when(gg >= 2)
            def _():
                scat_desc(bp).wait()

            idx_desc(gg + 2, bp).start()
            idx_desc(gg, u).wait()
            pltpu.async_copy(ones_v, acc.at[dst_v.at[u]], sem_s.at[u],
                             add=True)

    for gg in (CH_MAIN, CH_MAIN + 1):
        b = gg % NBUF
        scat_desc((gg + 2) % NBUF).wait()
        idx_desc(gg, b).wait()
        pltpu.async_copy(ones_v, acc.at[dst_v.at[b]], sem_s.at[b], add=True)
    scat_desc(CH_MAIN % NBUF).wait()
    scat_desc((CH_MAIN + 1) % NBUF).wait()

    @pl.when(wid < CH_REM)
    def _rem():
        base = (CH_FULL * NW + wid) * CHUNK
        pltpu.sync_copy(dst_e.at[pl.ds(base, CHUNK)], dst_v.at[0])
        pltpu.sync_copy(ones_v, acc.at[dst_v.at[0]], add=True)

    plsc.subcore_barrier()
    pltpu.sync_copy(acc.at[pl.ds(s * ROWS_T, ROWS_T)],
                    deg_out.at[c, pl.ds(s * ROWS_T, ROWS_T)])


NSPL = 4
SUB = CHUNK // NSPL


@functools.partial(
    pl.kernel,
    out_type=jax.ShapeDtypeStruct((NC, NPAD, D_IN), jnp.float32),
    mesh=_mesh,
    scratch_types=[
        pltpu.VMEM((NBUF, CHUNK), jnp.int32),
        pltpu.VMEM((NBUF, CHUNK), jnp.int32),
        pltpu.VMEM((2, CHUNK, D_IN), jnp.float32),
        pltpu.VMEM_SHARED((NPAD, D_IN), jnp.float32),
        pltpu.SemaphoreType.DMA((NBUF,)),
        pltpu.SemaphoreType.DMA((2,)),
        pltpu.SemaphoreType.DMA((2,)),
    ],
)
def _sc_agg1(rows_hbm, src_e, dst_e, agg_out, src_v, dst_v, rows_v, acc,
             sem_i, sem_g, sem_s):
    c = lax.axis_index("c")
    s = lax.axis_index("s")
    wid = s * NC + c

    pltpu.sync_copy(rows_hbm.at[pl.ds(s * ROWS_T, ROWS_T)],
                    acc.at[pl.ds(s * ROWS_T, ROWS_T)])
    plsc.subcore_barrier()

    def idx_descs(g, b):
        base = (g * NW + wid) * CHUNK
        return (pltpu.make_async_copy(src_e.at[pl.ds(base, CHUNK)],
                                      src_v.at[b], sem_i.at[b]),
                pltpu.make_async_copy(dst_e.at[pl.ds(base, CHUNK)],
                                      dst_v.at[b], sem_i.at[b]))

    def sub_gather_descs(ib, rb):
        return tuple(
            pltpu.make_async_copy(
                rows_hbm.at[src_v.at[ib, pl.ds(h * SUB, SUB)]],
                rows_v.at[rb, pl.ds(h * SUB, SUB)],
                sem_g.at[rb])
            for h in range(NSPL))

    def scat_desc(ib, rb):
        return pltpu.make_async_copy(rows_v.at[rb], acc.at[dst_v.at[ib]],
                                     sem_s.at[rb])

    for d in idx_descs(0, 0):
        d.start()
    for d in idx_descs(1, 1):
        d.start()

    @pl.loop(0, CH_MAIN, step=NBUF)
    def _pipe(g):
        for u in range(NBUF):
            gg = g + u
            rb = u % 2
            ip = (u + 2) % NBUF

            @pl.when(gg >= 2)
            def _():
                scat_desc(ip, rb).wait()

            for d in idx_descs(gg + 2, ip):
                d.start()
            for d in idx_descs(gg, u):
                d.wait()
            for d in sub_gather_descs(u, rb):
                d.start()
            for d in sub_gather_descs(u, rb):
                d.wait()
            pltpu.async_copy(rows_v.at[rb], acc.at[dst_v.at[u]],
                             sem_s.at[rb], add=True)

    for gg in (CH_MAIN, CH_MAIN + 1):
        ib = gg % NBUF
        rb = gg % 2
        scat_desc((gg + 2) % NBUF, rb).wait()
        for d in idx_descs(gg, ib):
            d.wait()
        for d in sub_gather_descs(ib, rb):
            d.start()
        for d in sub_gather_descs(ib, rb):
            d.wait()
        pltpu.async_copy(rows_v.at[rb], acc.at[dst_v.at[ib]], sem_s.at[rb],
                         add=True)
    scat_desc(CH_MAIN % NBUF, CH_MAIN % 2).wait()
    scat_desc((CH_MAIN + 1) % NBUF, (CH_MAIN + 1) % 2).wait()

    @pl.when(wid < CH_REM)
    def _rem():
        base = (CH_FULL * NW + wid) * CHUNK
        pltpu.sync_copy(src_e.at[pl.ds(base, CHUNK)], src_v.at[0])
        pltpu.sync_copy(dst_e.at[pl.ds(base, CHUNK)], dst_v.at[0])
        pltpu.sync_copy(rows_hbm.at[src_v.at[0]], rows_v.at[0])
        pltpu.sync_copy(rows_v.at[0], acc.at[dst_v.at[0]], add=True)

    plsc.subcore_barrier()
    pltpu.sync_copy(acc.at[pl.ds(s * ROWS_T, ROWS_T)],
                    agg_out.at[c, pl.ds(s * ROWS_T, ROWS_T)])


NB2 = 6
assert CH_FULL % NB2 == 0


@functools.partial(
    pl.kernel,
    out_type=(jax.ShapeDtypeStruct((NC, NPAD), jnp.float32),
              jax.ShapeDtypeStruct((NC, NPAD), jnp.float32)),
    mesh=_mesh,
    scratch_types=[
        pltpu.VMEM((NB2, CHUNK), jnp.int32),
        pltpu.VMEM((NB2, CHUNK), jnp.int32),
        pltpu.VMEM((NB2, CHUNK), jnp.float32),
        pltpu.VMEM((NB2, CHUNK), jnp.float32),
        pltpu.VMEM_SHARED((NPAD,), jnp.float32),
        pltpu.VMEM_SHARED((NPAD,), jnp.float32),
        pltpu.SemaphoreType.DMA((NB2,)),
        pltpu.SemaphoreType.DMA((NB2,)),
        pltpu.SemaphoreType.DMA((NB2,)),
    ],
)
def _sc_agg2(za, zb, src_e, dst_e, outa, outb, src_v, dst_v, ea_v, eb_v,
             acca, accb, sem_i, sem_g, sem_s):
    c = lax.axis_index("c")
    s = lax.axis_index("s")
    wid = s * NC + c

    pltpu.sync_copy(za.at[pl.ds(s * ROWS_T, ROWS_T)],
                    acca.at[pl.ds(s * ROWS_T, ROWS_T)])
    pltpu.sync_copy(zb.at[pl.ds(s * ROWS_T, ROWS_T)],
                    accb.at[pl.ds(s * ROWS_T, ROWS_T)])
    plsc.subcore_barrier()

    def idx_descs(g, b):
        base = (g * NW + wid) * CHUNK
        return (pltpu.make_async_copy(src_e.at[pl.ds(base, CHUNK)],
                                      src_v.at[b], sem_i.at[b]),
                pltpu.make_async_copy(dst_e.at[pl.ds(base, CHUNK)],
                                      dst_v.at[b], sem_i.at[b]))

    def gather_descs(b):
        return (pltpu.make_async_copy(za.at[src_v.at[b]], ea_v.at[b],
                                      sem_g.at[b]),
                pltpu.make_async_copy(zb.at[src_v.at[b]], eb_v.at[b],
                                      sem_g.at[b]))

    def scat_descs(b):
        return (pltpu.make_async_copy(ea_v.at[b], acca.at[dst_v.at[b]],
                                      sem_s.at[b]),
                pltpu.make_async_copy(eb_v.at[b], accb.at[dst_v.at[b]],
                                      sem_s.at[b]))

    def scat_start(b):
        pltpu.async_copy(ea_v.at[b], acca.at[dst_v.at[b]], sem_s.at[b],
                         add=True)
        pltpu.async_copy(eb_v.at[b], accb.at[dst_v.at[b]], sem_s.at[b],
                         add=True)

    for d in idx_descs(0, 0):
        d.start()
    for d in idx_descs(1, 1):
        d.start()

    @pl.loop(0, CH_FULL, step=NB2)
    def _pipe(g):
        for u in range(NB2):
            gg = g + u
            bp = (u + 2) % NB2
            bm = (u - 2) % NB2

            @pl.when(gg >= 4)
            def _():
                for d in scat_descs(bp):
                    d.wait()

            @pl.when(gg + 2 < CH_FULL)
            def _():
                for d in idx_descs(gg + 2, bp):
                    d.start()
            for d in idx_descs(gg, u):
                d.wait()
            for d in gather_descs(u):
                d.start()

            @pl.when(gg >= 2)
            def _():
                for d in gather_descs(bm):
                    d.wait()
                scat_start(bm)

    for gg in (CH_FULL - 2, CH_FULL - 1):
        b = gg % NB2
        for d in gather_descs(b):
            d.wait()
        scat_start(b)
    for gg in range(CH_FULL - 4, CH_FULL):
        for d in scat_descs(gg % NB2):
            d.wait()

    @pl.when(wid < CH_REM)
    def _rem():
        base = (CH_FULL * NW + wid) * CHUNK
        pltpu.sync_copy(src_e.at[pl.ds(base, CHUNK)], src_v.at[0])
        pltpu.sync_copy(dst_e.at[pl.ds(base, CHUNK)], dst_v.at[0])
        pltpu.sync_copy(za.at[src_v.at[0]], ea_v.at[0])
        pltpu.sync_copy(zb.at[src_v.at[0]], eb_v.at[0])
        pltpu.sync_copy(ea_v.at[0], acca.at[dst_v.at[0]], add=True)
        pltpu.sync_copy(eb_v.at[0], accb.at[dst_v.at[0]], add=True)

    plsc.subcore_barrier()
    pltpu.sync_copy(acca.at[pl.ds(s * ROWS_T, ROWS_T)],
                    outa.at[c, pl.ds(s * ROWS_T, ROWS_T)])
    pltpu.sync_copy(accb.at[pl.ds(s * ROWS_T, ROWS_T)],
                    outb.at[c, pl.ds(s * ROWS_T, ROWS_T)])


def _tc_scale_body(dega_ref, degb_ref, x_ref, xs_ref, dis_ref):
    deg = dega_ref[...] + degb_ref[...] + 1.0
    dis = lax.rsqrt(deg)
    dis_ref[...] = dis
    xs_ref[...] = x_ref[...] * dis


def _tc_scale(dega, degb, x):
    return pl.pallas_call(
        _tc_scale_body,
        out_shape=(
            jax.ShapeDtypeStruct((NPAD, D_IN), jnp.float32),
            jax.ShapeDtypeStruct((NPAD, 1), jnp.float32),
        ),
    )(dega, degb, x)


_RB = 2048


def _tc_mlp_body(acc_ref, xs_ref, dis_ref, w1_ref, b1_ref, w2_ref, z2s_ref):
    dis = dis_ref[...]
    a = (acc_ref[0] + acc_ref[1] - xs_ref[...]) * dis
    h = jnp.maximum(
        jnp.dot(a, w1_ref[...], preferred_element_type=jnp.float32)
        + b1_ref[...], 0.0)
    z2 = jnp.dot(h, w2_ref[...], preferred_element_type=jnp.float32)
    z2s_ref[...] = z2 * dis


def _tc_mlp(acc, xs, dis, W1, b1, W2):
    grid = (NPAD // _RB,)
    return pl.pallas_call(
        _tc_mlp_body,
        grid=grid,
        in_specs=[
            pl.BlockSpec((NC, _RB, D_IN), lambda i: (0, i, 0)),
            pl.BlockSpec((_RB, D_IN), lambda i: (i, 0)),
            pl.BlockSpec((_RB, 1), lambda i: (i, 0)),
            pl.BlockSpec((D_IN, D_HID), lambda i: (0, 0)),
            pl.BlockSpec((1, D_HID), lambda i: (0, 0)),
            pl.BlockSpec((D_HID, D_OUT), lambda i: (0, 0)),
        ],
        out_specs=pl.BlockSpec((_RB, D_OUT), lambda i: (i, 0)),
        out_shape=jax.ShapeDtypeStruct((NPAD, D_OUT), jnp.float32),
    )(acc, xs, dis, W1, b1, W2)


def _tc_final_body(a0_ref, a1_ref, b0_ref, b1c_ref, za_ref, zb_ref,
                   dis_ref, b2_ref, oa_ref, ob_ref):
    dis = dis_ref[...]
    oa_ref[...] = ((a0_ref[...] + a1_ref[...] - za_ref[...]) * dis
                   + b2_ref[0:1, 0:1])
    ob_ref[...] = ((b0_ref[...] + b1c_ref[...] - zb_ref[...]) * dis
                   + b2_ref[0:1, 1:2])


def _tc_final(acc2a, acc2b, za, zb, dis, b2):
    return pl.pallas_call(
        _tc_final_body,
        out_shape=(jax.ShapeDtypeStruct((N, 1), jnp.float32),
                   jax.ShapeDtypeStruct((N, 1), jnp.float32)),
    )(acc2a[0, :N, None], acc2a[1, :N, None],
      acc2b[0, :N, None], acc2b[1, :N, None],
      za[:N, None], zb[:N, None], dis[:N], b2.reshape(1, D_OUT))


def kernel(x, edge_index, W1, b1, W2, b2):
    ei = edge_index.astype(jnp.int32)
    src_e = ei[0]
    dst_e = ei[1]
    xp = jnp.pad(x, ((0, NPAD - N), (0, 0)))
    deg2 = _sc_degree(dst_e)
    xs, dis = _tc_scale(deg2[0, :, None], deg2[1, :, None], xp)
    acc = _sc_agg1(xs, src_e, dst_e)
    z2s = _tc_mlp(acc, xs, dis, W1, b1.reshape(1, D_HID), W2)
    acc2a, acc2b = _sc_agg2(z2s[:, 0], z2s[:, 1], src_e, dst_e)
    oa, ob = _tc_final(acc2a, acc2b, z2s[:, 0], z2s[:, 1], dis, b2)
    return jnp.concatenate([oa, ob], axis=1)
